# Initial kernel scaffold; baseline (speedup 1.0000x reference)
import jax
import jax.numpy as jnp
from jax.experimental import pallas as pl


def kernel(b_state, b_val, init_state, init_val, bk_Wq, bk_Wk, kb_Wq, kb_Wk, pp_Wq, pp_Wk, kv_g, kv_b, br_g, br_b, pn_g, pn_b):
    raise NotImplementedError("write your pallas kernel here")


# speedup vs baseline: 23.3642x; 23.3642x over previous
import functools
import jax
import jax.numpy as jnp
from jax.experimental import pallas as pl
from jax.experimental.pallas import tpu as pltpu

DIM = 768
N = 2048
RANK = 64
TOPK = 16
BLK = 256
NT = N // BLK
F32 = jnp.float32
NEG = -3.4e38


def _ln(x, g, b):
    m = jnp.mean(x, axis=-1, keepdims=True)
    xc = x - m
    v = jnp.mean(xc * xc, axis=-1, keepdims=True)
    return xc / jnp.sqrt(v + 1e-5) * g + b


def _softplus(x):
    return jnp.maximum(x, 0.0) + jnp.log1p(jnp.exp(-jnp.abs(x)))


def _dirs(x):
    n = jnp.sqrt(jnp.sum(x * x, axis=-1, keepdims=True)) + 1e-6
    return x / n


def _st4(x):
    return x.reshape(x.shape[0], NT, 1, BLK)


_SPEC_ST = pl.BlockSpec((None, None, 1, BLK), lambda i, t: (i, t, 0, 0))
_SPEC_VAL = pl.BlockSpec((None, BLK, DIM), lambda i, t: (i, t, 0))
_SPEC_QK = pl.BlockSpec((None, BLK, RANK), lambda i, t: (i, t, 0))
_SPEC_W = pl.BlockSpec((DIM, RANK), lambda i, t: (0, 0))
_SPEC_GB = pl.BlockSpec((1, DIM), lambda i, t: (0, 0))


def _kstate_body(s_ref, o_ref):
    s = s_ref[...]
    a = jnp.abs(s)
    mx = jnp.max(a, axis=-1, keepdims=True)
    e = jnp.exp(a - mx)
    o_ref[...] = jnp.sign(s) * e / jnp.sum(e, axis=-1, keepdims=True)


def _kstate(init_state):
    return pl.pallas_call(
        _kstate_body,
        out_shape=jax.ShapeDtypeStruct((1, N), F32),
    )(init_state.reshape(1, N))


def _prep1_body(bval_ref, bstate_ref, ival_ref, wq_ref, wk_ref, g_ref, b_ref,
                q_ref, k_ref, dirs_ref, gate_ref, kval_ref):
    bv = bval_ref[...]
    iv = ival_ref[...]
    g = g_ref[...]
    b = b_ref[...]
    kv = _ln(iv, g, b)
    nk = _ln(kv, g, b)
    k_ref[...] = jnp.dot(nk, wk_ref[...], preferred_element_type=F32)
    q_ref[...] = jnp.dot(bv, wq_ref[...], preferred_element_type=F32)
    dirs_ref[...] = _dirs(bv)
    gate_ref[...] = _softplus(bstate_ref[...])
    kval_ref[...] = kv


def _prep1(b_val, b_state, init_val, wq, wk, g, b):
    B = b_val.shape[0]
    q, k, dirs, gate4, kval = pl.pallas_call(
        _prep1_body,
        grid=(B, NT),
        in_specs=[
            _SPEC_VAL,
            _SPEC_ST,
            pl.BlockSpec((BLK, DIM), lambda i, t: (t, 0)),
            _SPEC_W, _SPEC_W, _SPEC_GB, _SPEC_GB,
        ],
        out_specs=[_SPEC_QK, _SPEC_QK, _SPEC_VAL, _SPEC_ST,
                   pl.BlockSpec((BLK, DIM), lambda i, t: (t, 0))],
        out_shape=[
            jax.ShapeDtypeStruct((B, N, RANK), F32),
            jax.ShapeDtypeStruct((B, N, RANK), F32),
            jax.ShapeDtypeStruct((B, N, DIM), F32),
            jax.ShapeDtypeStruct((B, NT, 1, BLK), F32),
            jax.ShapeDtypeStruct((N, DIM), F32),
        ],
    )(b_val, _st4(b_state), init_val, wq, wk,
      g.reshape(1, DIM), b.reshape(1, DIM))
    return q, k, dirs, gate4, kval


def _prep2_body(rv_ref, rs_ref, wq_ref, wk_ref, g_ref, b_ref,
                q_ref, k_ref, dirs_ref, gate_ref):
    nv = _ln(rv_ref[...], g_ref[...], b_ref[...])
    q_ref[...] = jnp.dot(nv, wq_ref[...], preferred_element_type=F32)
    k_ref[...] = jnp.dot(nv, wk_ref[...], preferred_element_type=F32)
    dirs_ref[...] = _dirs(nv)
    gate_ref[...] = _softplus(rs_ref[...])


def _prep2(routed_val, routed_state4, wq, wk, g, b):
    B = routed_val.shape[0]
    return pl.pallas_call(
        _prep2_body,
        grid=(B, NT),
        in_specs=[_SPEC_VAL, _SPEC_ST, _SPEC_W, _SPEC_W, _SPEC_GB, _SPEC_GB],
        out_specs=[_SPEC_QK, _SPEC_QK, _SPEC_VAL, _SPEC_ST],
        out_shape=[
            jax.ShapeDtypeStruct((B, N, RANK), F32),
            jax.ShapeDtypeStruct((B, N, RANK), F32),
            jax.ShapeDtypeStruct((B, N, DIM), F32),
            jax.ShapeDtypeStruct((B, NT, 1, BLK), F32),
        ],
    )(routed_val, routed_state4, wq, wk, g.reshape(1, DIM), b.reshape(1, DIM))


def _prep3_body(pv_ref, ps_ref, bv_ref, wq_ref, wk_ref,
                kvg_ref, kvb_ref, brg_ref, brb_ref,
                q_ref, k_ref, dirs_ref, gate_ref):
    nk2 = _ln(pv_ref[...], kvg_ref[...], kvb_ref[...])
    nb = _ln(bv_ref[...], brg_ref[...], brb_ref[...])
    q_ref[...] = jnp.dot(nk2, wq_ref[...], preferred_element_type=F32)
    k_ref[...] = jnp.dot(nb, wk_ref[...], preferred_element_type=F32)
    dirs_ref[...] = _dirs(nk2)
    gate_ref[...] = _softplus(ps_ref[...])


def _prep3(prop_val, prop_state4, b_val, wq, wk, kvg, kvb, brg, brb):
    B = prop_val.shape[0]
    return pl.pallas_call(
        _prep3_body,
        grid=(B, NT),
        in_specs=[_SPEC_VAL, _SPEC_ST, _SPEC_VAL, _SPEC_W, _SPEC_W,
                  _SPEC_GB, _SPEC_GB, _SPEC_GB, _SPEC_GB],
        out_specs=[_SPEC_QK, _SPEC_QK, _SPEC_VAL, _SPEC_ST],
        out_shape=[
            jax.ShapeDtypeStruct((B, N, RANK), F32),
            jax.ShapeDtypeStruct((B, N, RANK), F32),
            jax.ShapeDtypeStruct((B, N, DIM), F32),
            jax.ShapeDtypeStruct((B, NT, 1, BLK), F32),
        ],
    )(prop_val, prop_state4, b_val, wq, wk,
      kvg.reshape(1, DIM), kvb.reshape(1, DIM),
      brg.reshape(1, DIM), brb.reshape(1, DIM))


def _route_body_res(kd_ref, q_ref, gate_ref, dirs_ref, resv_ref, ress_ref,
                    eg_ref, eb_ref, so_ref, vo_ref):
    _route_core(kd_ref, q_ref, gate_ref, dirs_ref, so_ref, vo_ref,
                resv_ref, ress_ref, eg_ref, eb_ref)


def _route_body_nores(kd_ref, q_ref, gate_ref, dirs_ref, so_ref, vo_ref):
    _route_core(kd_ref, q_ref, gate_ref, dirs_ref, so_ref, vo_ref,
                None, None, None, None)


def _route_core(kd_ref, q_ref, gate_ref, dirs_ref, so_ref, vo_ref,
                resv_ref, ress_ref, eg_ref, eb_ref):
    kd = kd_ref[...]
    q = q_ref[...]
    s = jax.lax.dot_general(kd, q, (((1,), (1,)), ((), ())),
                            preferred_element_type=F32) * 0.125
    a = jnp.abs(s)
    m = jnp.max(a, axis=-1, keepdims=True)
    thr = m
    for _ in range(TOPK - 1):
        cur = jnp.where(a >= thr, NEG, a)
        thr = jnp.max(cur, axis=-1, keepdims=True)
    e = jnp.where(a >= thr, jnp.exp(a - m), 0.0)
    z = jnp.sum(e, axis=-1, keepdims=True)
    w = jnp.sign(s) * e * gate_ref[...] / z
    so = jnp.sum(w, axis=-1)
    dv = jnp.dot(w, dirs_ref[...], preferred_element_type=F32)
    if resv_ref is not None:
        so = so + ress_ref[...].reshape(-1)
        dv = _ln(resv_ref[...] + dv, eg_ref[...], eb_ref[...])
    so_ref[...] = so.reshape(1, BLK)
    vo_ref[...] = dv


def _route(kd, q, gate4, dirs, resv=None, ress=None, eg=None, eb=None,
           resv_batched=True, ress_batched=True):
    B = kd.shape[0]
    gate_row = gate4.reshape(B, 1, N)
    in_specs = [
        _SPEC_QK,
        pl.BlockSpec((None, N, RANK), lambda i, t: (i, 0, 0)),
        pl.BlockSpec((None, 1, N), lambda i, t: (i, 0, 0)),
        pl.BlockSpec((None, N, DIM), lambda i, t: (i, 0, 0)),
    ]
    args = [kd, q, gate_row, dirs]
    if resv is not None:
        if resv_batched:
            in_specs.append(_SPEC_VAL)
        else:
            in_specs.append(pl.BlockSpec((BLK, DIM), lambda i, t: (t, 0)))
        if ress_batched:
            in_specs.append(_SPEC_ST)
        else:
            in_specs.append(pl.BlockSpec((None, 1, BLK), lambda i, t: (t, 0, 0)))
        in_specs.append(_SPEC_GB)
        in_specs.append(_SPEC_GB)
        args += [resv, ress, eg.reshape(1, DIM), eb.reshape(1, DIM)]
        body = _route_body_res
    else:
        body = _route_body_nores
    so4, vo = pl.pallas_call(
        body,
        grid=(B, NT),
        in_specs=in_specs,
        out_specs=[_SPEC_ST, _SPEC_VAL],
        out_shape=[
            jax.ShapeDtypeStruct((B, NT, 1, BLK), F32),
            jax.ShapeDtypeStruct((B, N, DIM), F32),
        ],
    )(*args)
    return so4, vo


def kernel(b_state, b_val, init_state, init_val, bk_Wq, bk_Wk, kb_Wq, kb_Wk,
           pp_Wq, pp_Wk, kv_g, kv_b, br_g, br_b, pn_g, pn_b):
    B = b_state.shape[0]

    k_state1 = _kstate(init_state).reshape(NT, 1, BLK)

    q1, k1, dirs1, gate1, kval = _prep1(b_val, b_state, init_val,
                                        bk_Wq, bk_Wk, kv_g, kv_b)
    routed_state4, routed_val = _route(
        k1, q1, gate1, dirs1, resv=kval, ress=k_state1,
        eg=kv_g, eb=kv_b, resv_batched=False, ress_batched=False)

    q2, k2, dirs2, gate2 = _prep2(routed_val, routed_state4, pp_Wq, pp_Wk,
                                  pn_g, pn_b)
    prop_state4, prop_val = _route(
        k2, q2, gate2, dirs2, resv=routed_val, ress=routed_state4,
        eg=kv_g, eb=kv_b)

    q3, k3, dirs3, gate3 = _prep3(prop_val, prop_state4, b_val, kb_Wq, kb_Wk,
                                  kv_g, kv_b, br_g, br_b)
    bd_state4, bd_val = _route(k3, q3, gate3, dirs3)

    routed_state = routed_state4.reshape(B, N)
    prop_state = prop_state4.reshape(B, N)
    bd_state = bd_state4.reshape(B, N)
    return (routed_state, routed_val, prop_state, prop_val, bd_state, bd_val)
